# baseline (speedup 1.0000x reference)
import functools

import jax
import jax.numpy as jnp
from jax import lax
from jax.experimental import pallas as pl
from jax.experimental.pallas import tpu as pltpu
from jax.experimental.pallas import tpu_sc as plsc

_CHUNK = 240


def kernel(embed_user, embed_item):
    n, d = embed_user.shape
    info = plsc.get_sparse_core_info()
    nw = info.num_cores * info.num_subcores
    rows = (n // nw) // 8 * 8
    rem = n - nw * rows
    chunk = _CHUNK
    nchunks = rows // chunk
    assert rows % chunk == 0

    mesh = plsc.VectorSubcoreMesh(core_axis_name="c", subcore_axis_name="s")

    @functools.partial(
        pl.kernel,
        mesh=mesh,
        out_type=jax.ShapeDtypeStruct((2, n, d), embed_user.dtype),
        scratch_types=[
            pltpu.VMEM((2, chunk, d), jnp.float32),
            pltpu.VMEM((rem, d), jnp.float32),
            pltpu.SemaphoreType.DMA,
            pltpu.SemaphoreType.DMA,
            pltpu.SemaphoreType.DMA,
            pltpu.SemaphoreType.DMA,
        ],
    )
    def copy_tables(user_hbm, item_hbm, out_hbm, buf, tail_buf,
                    sem_in0, sem_in1, sem_out0, sem_out1):
        wid = lax.axis_index("s") * info.num_cores + lax.axis_index("c")
        base = wid * rows
        srcs = (user_hbm, item_hbm)
        sem_in = (sem_in0, sem_in1)
        sem_out = (sem_out0, sem_out1)

        def src_dst(k):
            t, c = divmod(k, nchunks)
            lo = base + c * chunk
            return (srcs[t].at[pl.ds(lo, chunk)],
                    out_hbm.at[t, pl.ds(lo, chunk)])

        total = 2 * nchunks
        loads = [None] * total
        stores = [None] * total
        for k in range(total):
            s, o = src_dst(k)
            loads[k] = pltpu.make_async_copy(s, buf.at[k % 2], sem_in[k % 2])
            stores[k] = pltpu.make_async_copy(buf.at[k % 2], o, sem_out[k % 2])

        loads[0].start()
        for k in range(total):
            if k + 1 < total:
                if k >= 1:
                    stores[k - 1].wait()
                loads[k + 1].start()
            loads[k].wait()
            stores[k].start()
        stores[total - 2].wait()
        stores[total - 1].wait()

        if rem:
            tail = nw * rows

            @pl.when(wid == 0)
            def _():
                for t in range(2):
                    lt = pltpu.make_async_copy(
                        srcs[t].at[pl.ds(tail, rem)], tail_buf, sem_in[0])
                    st = pltpu.make_async_copy(
                        tail_buf, out_hbm.at[t, pl.ds(tail, rem)], sem_out[0])
                    lt.start()
                    lt.wait()
                    st.start()
                    st.wait()

    return copy_tables(embed_user, embed_item)


# speedup vs baseline: 1.3359x; 1.3359x over previous
import jax
import jax.numpy as jnp
from jax.experimental import pallas as pl
from jax.experimental.pallas import tpu as pltpu

_BLOCK_ROWS = 2000


def _copy_body(user_ref, item_ref, out_ref):
    out_ref[0] = user_ref[...]
    out_ref[1] = item_ref[...]


def kernel(embed_user, embed_item):
    n, d = embed_user.shape
    bn = _BLOCK_ROWS if n % _BLOCK_ROWS == 0 else n
    grid = (n // bn,)
    return pl.pallas_call(
        _copy_body,
        grid=grid,
        in_specs=[
            pl.BlockSpec((bn, d), lambda j: (j, 0)),
            pl.BlockSpec((bn, d), lambda j: (j, 0)),
        ],
        out_specs=pl.BlockSpec((2, bn, d), lambda j: (0, j, 0)),
        out_shape=jax.ShapeDtypeStruct((2, n, d), embed_user.dtype),
    )(embed_user, embed_item)


# speedup vs baseline: 1.4967x; 1.1204x over previous
import jax
import jax.numpy as jnp
from jax.experimental import pallas as pl
from jax.experimental.pallas import tpu as pltpu

_BLOCK_ROWS = 10000


def _copy_body(user_ref, item_ref, out_ref):
    out_ref[0] = user_ref[...]
    out_ref[1] = item_ref[...]


def kernel(embed_user, embed_item):
    n, d = embed_user.shape
    bn = _BLOCK_ROWS if n % _BLOCK_ROWS == 0 else n
    grid = (n // bn,)
    return pl.pallas_call(
        _copy_body,
        grid=grid,
        in_specs=[
            pl.BlockSpec((bn, d), lambda j: (j, 0)),
            pl.BlockSpec((bn, d), lambda j: (j, 0)),
        ],
        out_specs=pl.BlockSpec((2, bn, d), lambda j: (0, j, 0)),
        out_shape=jax.ShapeDtypeStruct((2, n, d), embed_user.dtype),
    )(embed_user, embed_item)
